# baseline (speedup 1.0000x reference)
import jax
import jax.numpy as jnp
from jax import lax
from jax.experimental import pallas as pl
from jax.experimental.pallas import tpu as pltpu
from jax.experimental.pallas import tpu_sc as plsc

_SIZE = 1024
_HALF = 512
_SCW = _SIZE - _HALF
_ROWS = 32768
_N = 16384
_NC, _NS = 2, 16
_NW = _NC * _NS
_CHUNK = 16
_PERW = _N // _NW
_NCHUNK = _PERW // _CHUNK
_JCH = _SCW // 16
_UNROLL = 4
_MV_BLOCK = 2048


_DCH = 32
_ND = _PERW // _DCH
_RING = 4


def _gdot_body(tab_hbm, idx_hbm, w_hbm, out_hbm,
               idx_v, w_v, rows_v, vals_v, sems):
    wid = lax.axis_index("s") * _NC + lax.axis_index("c")
    pltpu.sync_copy(idx_hbm.at[pl.ds(wid * _PERW, _PERW)], idx_v)
    pltpu.sync_copy(w_hbm, w_v)

    lanes = lax.iota(jnp.int32, 16)
    rot_idx = [(lanes + sh) % 16 for sh in (8, 4, 2, 1)]
    gdn = lax.GatherDimensionNumbers(offset_dims=(), collapsed_slice_dims=(0,),
                                     start_index_map=(0,))

    def hsum(v):
        for idx in rot_idx:
            v = v + lax.gather(v, idx[:, None], gdn, slice_sizes=(1,),
                               mode=lax.GatherScatterMode.PROMISE_IN_BOUNDS)
        return v

    def _desc(d, buf):
        dd = jnp.minimum(d, _ND - 1)
        return (tab_hbm.at[idx_v.at[pl.ds(dd * _DCH, _DCH)], pl.ds(_HALF, _SCW)],
                rows_v.at[buf], sems.at[buf])

    def gather(d, buf):
        src, dst, sem = _desc(d, buf)
        pltpu.async_copy(src, dst, sem)

    def wait(d, buf):
        src, dst, sem = _desc(d, buf)
        pltpu.make_async_copy(src, dst, sem).wait()

    def dot_chunk(c, rows_ref, r0):
        def jbody(j, accs):
            for u in range(_UNROLL):
                jj = j * _UNROLL + u
                wj = w_v[pl.ds(_HALF + jj * 16, 16)]
                accs = tuple(
                    accs[r] + rows_ref[r0 + r, pl.ds(jj * 16, 16)] * wj
                    for r in range(_CHUNK)
                )
            return accs

        accs = lax.fori_loop(
            0, _JCH // _UNROLL, jbody,
            tuple(jnp.zeros((16,), jnp.float32) for _ in range(_CHUNK)))
        v = jnp.zeros((16,), jnp.float32)
        for r in range(_CHUNK):
            v = jnp.where(lanes == r, hsum(accs[r]), v)
        vals_v[pl.ds(c * _CHUNK, _CHUNK)] = v

    for b in range(_RING):
        gather(b, b)

    def group(k, carry):
        d0 = k * _RING
        for b in range(_RING):
            d = d0 + b
            wait(d, b)
            dot_chunk(d * 2, rows_v.at[b], 0)
            dot_chunk(d * 2 + 1, rows_v.at[b], _CHUNK)
            gather(d + _RING, b)
        return carry

    lax.fori_loop(0, _ND // _RING, group, jnp.int32(0))
    for b in range(_RING):
        wait(_ND - 1, b)
    pltpu.sync_copy(vals_v, out_hbm.at[pl.ds(wid * _PERW, _PERW)])


def _sc_gather_dot(flat, idxd, w_hi):
    call = pl.kernel(
        _gdot_body,
        out_type=jax.ShapeDtypeStruct((_N,), jnp.float32),
        mesh=plsc.VectorSubcoreMesh(core_axis_name="c", subcore_axis_name="s"),
        scratch_types=[
            pltpu.VMEM((_PERW,), jnp.int32),
            pltpu.VMEM((_SIZE,), jnp.float32),
            pltpu.VMEM((_RING, _DCH, _SCW), jnp.float32),
            pltpu.VMEM((_PERW,), jnp.float32),
            pltpu.SemaphoreType.DMA((_RING,)),
        ],
    )
    return call(flat, idxd, w_hi)


_GROWS = _N // 128
_GPERW = _GROWS // _NW


def _scalar_gather_body(p_hbm, idx_hbm, out_hbm, idx_v, vals_v, sem):
    wid = lax.axis_index("s") * _NC + lax.axis_index("c")
    base = wid * _PERW
    pltpu.sync_copy(idx_hbm.at[pl.ds(base, _PERW)], idx_v)
    copies = [
        pltpu.async_copy(p_hbm.at[idx_v.at[pl.ds(j * 128, 128)]],
                         vals_v.at[pl.ds(j * 128, 128)], sem)
        for j in range(_PERW // 128)
    ]
    for c in copies:
        c.wait()
    pltpu.sync_copy(vals_v, out_hbm.at[pl.ds(base, _PERW)])


def _sc_scalar_gather(p_lo, idx):
    call = pl.kernel(
        _scalar_gather_body,
        out_type=jax.ShapeDtypeStruct((_N,), jnp.float32),
        mesh=plsc.VectorSubcoreMesh(core_axis_name="c", subcore_axis_name="s"),
        scratch_types=[
            pltpu.VMEM((_PERW,), jnp.int32),
            pltpu.VMEM((_PERW,), jnp.float32),
            pltpu.SemaphoreType.DMA,
        ],
    )
    return call(p_lo, idx)


def _matvec_body(x_ref, w_ref, b_ref, o_ref):
    o_ref[...] = jnp.sum(x_ref[...] * w_ref[...], axis=1) + b_ref[0]


def _matvec(flat, w_lo, b):
    return pl.pallas_call(
        _matvec_body,
        grid=(_ROWS // _MV_BLOCK,),
        in_specs=[
            pl.BlockSpec((_MV_BLOCK, _HALF), lambda i: (i, 0)),
            pl.BlockSpec((1, _HALF), lambda i: (0, 0)),
            pl.BlockSpec(memory_space=pltpu.SMEM),
        ],
        out_specs=pl.BlockSpec((_MV_BLOCK,), lambda i: (i,)),
        out_shape=jax.ShapeDtypeStruct((_ROWS,), jnp.float32),
    )(flat, w_lo, b)


def _finish_body(g_ref, hi_ref, t_ref, f_ref, l_ref):
    f = g_ref[...] + hi_ref[...]
    t = t_ref[...]
    f_ref[...] = f
    val = jnp.sum(jnp.maximum(f, 0.0) - f * t + jnp.log1p(jnp.exp(-jnp.abs(f))))
    l_ref[...] = val.reshape(1, 1)


def _finish(g, hi, targets):
    return pl.pallas_call(
        _finish_body,
        out_shape=(jax.ShapeDtypeStruct((_N,), jnp.float32),
                   jax.ShapeDtypeStruct((1, 1), jnp.float32)),
    )(g, hi, targets)


def kernel(rnn_output, indices, targets, W, b):
    flat = rnn_output.reshape(_ROWS, _SIZE)
    idx = indices.astype(jnp.int32)
    w_vec = W.reshape(_SIZE)
    p_lo = _matvec(flat, W, b)
    hi = _sc_gather_dot(flat, idx, w_vec)
    g = _sc_scalar_gather(p_lo, idx)
    final, loss = _finish(g, hi, targets)
    return final, loss.reshape(())


# speedup vs baseline: 1.2511x; 1.2511x over previous
import jax
import jax.numpy as jnp
from jax import lax
from jax.experimental import pallas as pl
from jax.experimental.pallas import tpu as pltpu
from jax.experimental.pallas import tpu_sc as plsc

_SIZE = 1024
_HALF = 512
_SCW = _SIZE - _HALF
_ROWS = 32768
_N = 16384
_NC, _NS = 2, 16
_NW = _NC * _NS
_CHUNK = 16
_PERW = _N // _NW
_NCHUNK = _PERW // _CHUNK
_JCH = _SCW // 16
_UNROLL = 4
_MV_BLOCK = 2048


_DCH = 32
_ND = _PERW // _DCH
_RING = 4


def _gdot_body(tab_hbm, idx_hbm, w_hbm, out_hbm,
               idx_v, w_v, rows_v, vals_v, sems):
    wid = lax.axis_index("s") * _NC + lax.axis_index("c")
    pltpu.sync_copy(idx_hbm.at[pl.ds(wid * _ND, _ND)], idx_v)
    pltpu.sync_copy(w_hbm, w_v)

    lanes = lax.iota(jnp.int32, 16)
    rot_idx = [(lanes + sh) % 16 for sh in (8, 4, 2, 1)]
    gdn = lax.GatherDimensionNumbers(offset_dims=(), collapsed_slice_dims=(0,),
                                     start_index_map=(0,))

    def hsum(v):
        for idx in rot_idx:
            v = v + lax.gather(v, idx[:, None], gdn, slice_sizes=(1,),
                               mode=lax.GatherScatterMode.PROMISE_IN_BOUNDS)
        return v

    def _desc(d, buf):
        dd = jnp.minimum(d, _ND - 1)
        return (tab_hbm.at[idx_v.at[dd], pl.ds(_HALF, _SCW)],
                rows_v.at[buf], sems.at[buf])

    def gather(d, buf):
        src, dst, sem = _desc(d, buf)
        pltpu.async_copy(src, dst, sem)

    def wait(d, buf):
        src, dst, sem = _desc(d, buf)
        pltpu.make_async_copy(src, dst, sem).wait()

    def dot_chunk(c, rows_ref, r0):
        def jbody(j, accs):
            for u in range(_UNROLL):
                jj = j * _UNROLL + u
                wj = w_v[pl.ds(jj * 16, 16)]
                accs = tuple(
                    accs[r] + rows_ref[r0 + r, pl.ds(jj * 16, 16)] * wj
                    for r in range(_CHUNK)
                )
            return accs

        accs = lax.fori_loop(
            0, _JCH // _UNROLL, jbody,
            tuple(jnp.zeros((16,), jnp.float32) for _ in range(_CHUNK)))
        v = jnp.zeros((16,), jnp.float32)
        for r in range(_CHUNK):
            v = jnp.where(lanes == r, hsum(accs[r]), v)
        vals_v[pl.ds(c * _CHUNK, _CHUNK)] = v

    for b in range(_RING):
        gather(b, b)

    def group(k, carry):
        d0 = k * _RING
        for b in range(_RING):
            d = d0 + b
            wait(d, b)
            dot_chunk(d * 2, rows_v.at[b], 0)
            dot_chunk(d * 2 + 1, rows_v.at[b], _CHUNK)
            gather(d + _RING, b)
        return carry

    lax.fori_loop(0, _ND // _RING, group, jnp.int32(0))
    for b in range(_RING):
        wait(_ND - 1, b)
    pltpu.sync_copy(vals_v, out_hbm.at[pl.ds(wid * _PERW, _PERW)])


def _sc_gather_dot(flat, idxd, w_hi):
    call = pl.kernel(
        _gdot_body,
        out_type=jax.ShapeDtypeStruct((_N,), jnp.float32),
        mesh=plsc.VectorSubcoreMesh(core_axis_name="c", subcore_axis_name="s"),
        scratch_types=[
            pltpu.VMEM((_ND, _DCH), jnp.int32),
            pltpu.VMEM((_SCW,), jnp.float32),
            pltpu.VMEM((_RING, _DCH, _SCW), jnp.float32),
            pltpu.VMEM((_PERW,), jnp.float32),
            pltpu.SemaphoreType.DMA((_RING,)),
        ],
    )
    return call(flat, idxd, w_hi)


_GROWS = _N // 128
_GPERW = _GROWS // _NW


def _scalar_gather_body(p_hbm, idx_hbm, out_hbm, idx_v, vals_v, sem):
    wid = lax.axis_index("s") * _NC + lax.axis_index("c")
    base = wid * _GPERW
    pltpu.sync_copy(idx_hbm.at[pl.ds(base, _GPERW)], idx_v)
    copies = [
        pltpu.async_copy(p_hbm.at[idx_v.at[j]], vals_v.at[j], sem)
        for j in range(_GPERW)
    ]
    for c in copies:
        c.wait()
    pltpu.sync_copy(vals_v, out_hbm.at[pl.ds(base, _GPERW)])


def _sc_scalar_gather(p_lo, idx128):
    call = pl.kernel(
        _scalar_gather_body,
        out_type=jax.ShapeDtypeStruct((_GROWS, 128), jnp.float32),
        mesh=plsc.VectorSubcoreMesh(core_axis_name="c", subcore_axis_name="s"),
        scratch_types=[
            pltpu.VMEM((_GPERW, 128), jnp.int32),
            pltpu.VMEM((_GPERW, 128), jnp.float32),
            pltpu.SemaphoreType.DMA,
        ],
    )
    return call(p_lo, idx128)


def _matvec_body(x_ref, w_ref, b_ref, o_ref):
    o_ref[...] = jnp.sum(x_ref[...] * w_ref[...], axis=1) + b_ref[0]


def _matvec(flat, w_lo, b):
    return pl.pallas_call(
        _matvec_body,
        grid=(_ROWS // _MV_BLOCK,),
        in_specs=[
            pl.BlockSpec((_MV_BLOCK, _HALF), lambda i: (i, 0)),
            pl.BlockSpec((1, _HALF), lambda i: (0, 0)),
            pl.BlockSpec(memory_space=pltpu.SMEM),
        ],
        out_specs=pl.BlockSpec((_MV_BLOCK,), lambda i: (i,)),
        out_shape=jax.ShapeDtypeStruct((_ROWS,), jnp.float32),
    )(flat, w_lo, b)


def _finish_body(g_ref, hi_ref, t_ref, f_ref, l_ref):
    f = g_ref[...].reshape(_N) + hi_ref[...]
    t = t_ref[...]
    f_ref[...] = f
    val = jnp.sum(jnp.maximum(f, 0.0) - f * t + jnp.log1p(jnp.exp(-jnp.abs(f))))
    l_ref[...] = val.reshape(1, 1)


def _finish(g, hi, targets):
    return pl.pallas_call(
        _finish_body,
        out_shape=(jax.ShapeDtypeStruct((_N,), jnp.float32),
                   jax.ShapeDtypeStruct((1, 1), jnp.float32)),
    )(g, hi, targets)


def kernel(rnn_output, indices, targets, W, b):
    flat = rnn_output.reshape(_ROWS, _SIZE)
    idx = indices.astype(jnp.int32)
    idxd = idx.reshape(_N // _DCH, _DCH)
    idx128 = idx.reshape(_GROWS, 128)
    w_lo = W[:, :_HALF]
    w_hi = W.reshape(_SIZE)[_HALF:]
    hi = _sc_gather_dot(flat, idxd, w_hi)
    p_lo = _matvec(flat, w_lo, b)
    g2d = _sc_scalar_gather(p_lo, idx128)
    final, loss = _finish(g2d, hi, targets)
    return final, loss.reshape(())


# speedup vs baseline: 1.2723x; 1.0170x over previous
import jax
import jax.numpy as jnp
from jax import lax
from jax.experimental import pallas as pl
from jax.experimental.pallas import tpu as pltpu
from jax.experimental.pallas import tpu_sc as plsc

_SIZE = 1024
_HALF = 384
_SCW = _SIZE - _HALF
_ROWS = 32768
_N = 16384
_NC, _NS = 2, 16
_NW = _NC * _NS
_CHUNK = 16
_PERW = _N // _NW
_NCHUNK = _PERW // _CHUNK
_JCH = _SCW // 16
_UNROLL = 4
_MV_BLOCK = 2048


_DCH = 32
_ND = _PERW // _DCH
_RING = 4


def _gdot_body(tab_hbm, idx_hbm, w_hbm, out_hbm,
               idx_v, w_v, rows_v, vals_v, sems):
    wid = lax.axis_index("s") * _NC + lax.axis_index("c")
    pltpu.sync_copy(idx_hbm.at[pl.ds(wid * _ND, _ND)], idx_v)
    pltpu.sync_copy(w_hbm, w_v)

    lanes = lax.iota(jnp.int32, 16)
    rot_idx = [(lanes + sh) % 16 for sh in (8, 4, 2, 1)]
    gdn = lax.GatherDimensionNumbers(offset_dims=(), collapsed_slice_dims=(0,),
                                     start_index_map=(0,))

    def hsum(v):
        for idx in rot_idx:
            v = v + lax.gather(v, idx[:, None], gdn, slice_sizes=(1,),
                               mode=lax.GatherScatterMode.PROMISE_IN_BOUNDS)
        return v

    def _desc(d, buf):
        dd = jnp.minimum(d, _ND - 1)
        return (tab_hbm.at[idx_v.at[dd], pl.ds(_HALF, _SCW)],
                rows_v.at[buf], sems.at[buf])

    def gather(d, buf):
        src, dst, sem = _desc(d, buf)
        pltpu.async_copy(src, dst, sem)

    def wait(d, buf):
        src, dst, sem = _desc(d, buf)
        pltpu.make_async_copy(src, dst, sem).wait()

    def dot_chunk(c, rows_ref, r0):
        def jbody(j, accs):
            for u in range(_UNROLL):
                jj = j * _UNROLL + u
                wj = w_v[pl.ds(jj * 16, 16)]
                accs = tuple(
                    accs[r] + rows_ref[r0 + r, pl.ds(jj * 16, 16)] * wj
                    for r in range(_CHUNK)
                )
            return accs

        accs = lax.fori_loop(
            0, _JCH // _UNROLL, jbody,
            tuple(jnp.zeros((16,), jnp.float32) for _ in range(_CHUNK)))
        v = jnp.zeros((16,), jnp.float32)
        for r in range(_CHUNK):
            v = jnp.where(lanes == r, hsum(accs[r]), v)
        vals_v[pl.ds(c * _CHUNK, _CHUNK)] = v

    for b in range(_RING):
        gather(b, b)

    def group(k, carry):
        d0 = k * _RING
        for b in range(_RING):
            d = d0 + b
            wait(d, b)
            dot_chunk(d * 2, rows_v.at[b], 0)
            dot_chunk(d * 2 + 1, rows_v.at[b], _CHUNK)
            gather(d + _RING, b)
        return carry

    lax.fori_loop(0, _ND // _RING, group, jnp.int32(0))
    for b in range(_RING):
        wait(_ND - 1, b)
    pltpu.sync_copy(vals_v, out_hbm.at[pl.ds(wid * _PERW, _PERW)])


def _sc_gather_dot(flat, idxd, w_hi):
    call = pl.kernel(
        _gdot_body,
        out_type=jax.ShapeDtypeStruct((_N,), jnp.float32),
        mesh=plsc.VectorSubcoreMesh(core_axis_name="c", subcore_axis_name="s"),
        scratch_types=[
            pltpu.VMEM((_ND, _DCH), jnp.int32),
            pltpu.VMEM((_SCW,), jnp.float32),
            pltpu.VMEM((_RING, _DCH, _SCW), jnp.float32),
            pltpu.VMEM((_PERW,), jnp.float32),
            pltpu.SemaphoreType.DMA((_RING,)),
        ],
    )
    return call(flat, idxd, w_hi)


_GROWS = _N // 128
_GPERW = _GROWS // _NW


def _scalar_gather_body(p_hbm, idx_hbm, out_hbm, idx_v, vals_v, sem):
    wid = lax.axis_index("s") * _NC + lax.axis_index("c")
    base = wid * _GPERW
    pltpu.sync_copy(idx_hbm.at[pl.ds(base, _GPERW)], idx_v)
    copies = [
        pltpu.async_copy(p_hbm.at[idx_v.at[j]], vals_v.at[j], sem)
        for j in range(_GPERW)
    ]
    for c in copies:
        c.wait()
    pltpu.sync_copy(vals_v, out_hbm.at[pl.ds(base, _GPERW)])


def _sc_scalar_gather(p_lo, idx128):
    call = pl.kernel(
        _scalar_gather_body,
        out_type=jax.ShapeDtypeStruct((_GROWS, 128), jnp.float32),
        mesh=plsc.VectorSubcoreMesh(core_axis_name="c", subcore_axis_name="s"),
        scratch_types=[
            pltpu.VMEM((_GPERW, 128), jnp.int32),
            pltpu.VMEM((_GPERW, 128), jnp.float32),
            pltpu.SemaphoreType.DMA,
        ],
    )
    return call(p_lo, idx128)


def _matvec_body(x_ref, w_ref, b_ref, o_ref):
    o_ref[...] = jnp.sum(x_ref[...] * w_ref[...], axis=1) + b_ref[0]


def _matvec(flat, w_lo, b):
    return pl.pallas_call(
        _matvec_body,
        grid=(_ROWS // _MV_BLOCK,),
        in_specs=[
            pl.BlockSpec((_MV_BLOCK, _HALF), lambda i: (i, 0)),
            pl.BlockSpec((1, _HALF), lambda i: (0, 0)),
            pl.BlockSpec(memory_space=pltpu.SMEM),
        ],
        out_specs=pl.BlockSpec((_MV_BLOCK,), lambda i: (i,)),
        out_shape=jax.ShapeDtypeStruct((_ROWS,), jnp.float32),
    )(flat, w_lo, b)


def _finish_body(g_ref, hi_ref, t_ref, f_ref, l_ref):
    f = g_ref[...].reshape(_N) + hi_ref[...]
    t = t_ref[...]
    f_ref[...] = f
    val = jnp.sum(jnp.maximum(f, 0.0) - f * t + jnp.log1p(jnp.exp(-jnp.abs(f))))
    l_ref[...] = val.reshape(1, 1)


def _finish(g, hi, targets):
    return pl.pallas_call(
        _finish_body,
        out_shape=(jax.ShapeDtypeStruct((_N,), jnp.float32),
                   jax.ShapeDtypeStruct((1, 1), jnp.float32)),
    )(g, hi, targets)


def kernel(rnn_output, indices, targets, W, b):
    flat = rnn_output.reshape(_ROWS, _SIZE)
    idx = indices.astype(jnp.int32)
    idxd = idx.reshape(_N // _DCH, _DCH)
    idx128 = idx.reshape(_GROWS, 128)
    w_lo = W[:, :_HALF]
    w_hi = W.reshape(_SIZE)[_HALF:]
    hi = _sc_gather_dot(flat, idxd, w_hi)
    p_lo = _matvec(flat, w_lo, b)
    g2d = _sc_scalar_gather(p_lo, idx128)
    final, loss = _finish(g2d, hi, targets)
    return final, loss.reshape(())


# speedup vs baseline: 1.2744x; 1.0016x over previous
import jax
import jax.numpy as jnp
from jax import lax
from jax.experimental import pallas as pl
from jax.experimental.pallas import tpu as pltpu
from jax.experimental.pallas import tpu_sc as plsc

_SIZE = 1024
_HALF = 256
_SCW = _SIZE - _HALF
_ROWS = 32768
_N = 16384
_NC, _NS = 2, 16
_NW = _NC * _NS
_CHUNK = 16
_PERW = _N // _NW
_NCHUNK = _PERW // _CHUNK
_JCH = _SCW // 16
_UNROLL = 4
_MV_BLOCK = 2048


_DCH = 32
_ND = _PERW // _DCH
_RING = 4


def _gdot_body(tab_hbm, idx_hbm, w_hbm, out_hbm,
               idx_v, w_v, rows_v, vals_v, sems):
    wid = lax.axis_index("s") * _NC + lax.axis_index("c")
    pltpu.sync_copy(idx_hbm.at[pl.ds(wid * _ND, _ND)], idx_v)
    pltpu.sync_copy(w_hbm, w_v)

    lanes = lax.iota(jnp.int32, 16)
    rot_idx = [(lanes + sh) % 16 for sh in (8, 4, 2, 1)]
    gdn = lax.GatherDimensionNumbers(offset_dims=(), collapsed_slice_dims=(0,),
                                     start_index_map=(0,))

    def hsum(v):
        for idx in rot_idx:
            v = v + lax.gather(v, idx[:, None], gdn, slice_sizes=(1,),
                               mode=lax.GatherScatterMode.PROMISE_IN_BOUNDS)
        return v

    def _desc(d, buf):
        dd = jnp.minimum(d, _ND - 1)
        return (tab_hbm.at[idx_v.at[dd], pl.ds(_HALF, _SCW)],
                rows_v.at[buf], sems.at[buf])

    def gather(d, buf):
        src, dst, sem = _desc(d, buf)
        pltpu.async_copy(src, dst, sem)

    def wait(d, buf):
        src, dst, sem = _desc(d, buf)
        pltpu.make_async_copy(src, dst, sem).wait()

    def dot_chunk(c, rows_ref, r0):
        def jbody(j, accs):
            for u in range(_UNROLL):
                jj = j * _UNROLL + u
                wj = w_v[pl.ds(jj * 16, 16)]
                accs = tuple(
                    accs[r] + rows_ref[r0 + r, pl.ds(jj * 16, 16)] * wj
                    for r in range(_CHUNK)
                )
            return accs

        accs = lax.fori_loop(
            0, _JCH // _UNROLL, jbody,
            tuple(jnp.zeros((16,), jnp.float32) for _ in range(_CHUNK)))
        v = jnp.zeros((16,), jnp.float32)
        for r in range(_CHUNK):
            v = jnp.where(lanes == r, hsum(accs[r]), v)
        vals_v[pl.ds(c * _CHUNK, _CHUNK)] = v

    for b in range(_RING):
        gather(b, b)

    def group(k, carry):
        d0 = k * _RING
        for b in range(_RING):
            d = d0 + b
            wait(d, b)
            dot_chunk(d * 2, rows_v.at[b], 0)
            dot_chunk(d * 2 + 1, rows_v.at[b], _CHUNK)
            gather(d + _RING, b)
        return carry

    lax.fori_loop(0, _ND // _RING, group, jnp.int32(0))
    for b in range(_RING):
        wait(_ND - 1, b)
    pltpu.sync_copy(vals_v, out_hbm.at[pl.ds(wid * _PERW, _PERW)])


def _sc_gather_dot(flat, idxd, w_hi):
    call = pl.kernel(
        _gdot_body,
        out_type=jax.ShapeDtypeStruct((_N,), jnp.float32),
        mesh=plsc.VectorSubcoreMesh(core_axis_name="c", subcore_axis_name="s"),
        scratch_types=[
            pltpu.VMEM((_ND, _DCH), jnp.int32),
            pltpu.VMEM((_SCW,), jnp.float32),
            pltpu.VMEM((_RING, _DCH, _SCW), jnp.float32),
            pltpu.VMEM((_PERW,), jnp.float32),
            pltpu.SemaphoreType.DMA((_RING,)),
        ],
    )
    return call(flat, idxd, w_hi)


_GROWS = _N // 128
_GPERW = _GROWS // _NW


def _scalar_gather_body(p_hbm, idx_hbm, out_hbm, idx_v, vals_v, sem):
    wid = lax.axis_index("s") * _NC + lax.axis_index("c")
    base = wid * _GPERW
    pltpu.sync_copy(idx_hbm.at[pl.ds(base, _GPERW)], idx_v)
    copies = [
        pltpu.async_copy(p_hbm.at[idx_v.at[j]], vals_v.at[j], sem)
        for j in range(_GPERW)
    ]
    for c in copies:
        c.wait()
    pltpu.sync_copy(vals_v, out_hbm.at[pl.ds(base, _GPERW)])


def _sc_scalar_gather(p_lo, idx128):
    call = pl.kernel(
        _scalar_gather_body,
        out_type=jax.ShapeDtypeStruct((_GROWS, 128), jnp.float32),
        mesh=plsc.VectorSubcoreMesh(core_axis_name="c", subcore_axis_name="s"),
        scratch_types=[
            pltpu.VMEM((_GPERW, 128), jnp.int32),
            pltpu.VMEM((_GPERW, 128), jnp.float32),
            pltpu.SemaphoreType.DMA,
        ],
    )
    return call(p_lo, idx128)


def _matvec_body(x_ref, w_ref, b_ref, o_ref):
    o_ref[...] = jnp.sum(x_ref[...] * w_ref[...], axis=1) + b_ref[0]


def _matvec(flat, w_lo, b):
    return pl.pallas_call(
        _matvec_body,
        grid=(_ROWS // _MV_BLOCK,),
        in_specs=[
            pl.BlockSpec((_MV_BLOCK, _HALF), lambda i: (i, 0)),
            pl.BlockSpec((1, _HALF), lambda i: (0, 0)),
            pl.BlockSpec(memory_space=pltpu.SMEM),
        ],
        out_specs=pl.BlockSpec((_MV_BLOCK,), lambda i: (i,)),
        out_shape=jax.ShapeDtypeStruct((_ROWS,), jnp.float32),
    )(flat, w_lo, b)


def _finish_body(g_ref, hi_ref, t_ref, f_ref, l_ref):
    f = g_ref[...].reshape(_N) + hi_ref[...]
    t = t_ref[...]
    f_ref[...] = f
    val = jnp.sum(jnp.maximum(f, 0.0) - f * t + jnp.log1p(jnp.exp(-jnp.abs(f))))
    l_ref[...] = val.reshape(1, 1)


def _finish(g, hi, targets):
    return pl.pallas_call(
        _finish_body,
        out_shape=(jax.ShapeDtypeStruct((_N,), jnp.float32),
                   jax.ShapeDtypeStruct((1, 1), jnp.float32)),
    )(g, hi, targets)


def kernel(rnn_output, indices, targets, W, b):
    flat = rnn_output.reshape(_ROWS, _SIZE)
    idx = indices.astype(jnp.int32)
    idxd = idx.reshape(_N // _DCH, _DCH)
    idx128 = idx.reshape(_GROWS, 128)
    w_lo = W[:, :_HALF]
    w_hi = W.reshape(_SIZE)[_HALF:]
    hi = _sc_gather_dot(flat, idxd, w_hi)
    p_lo = _matvec(flat, w_lo, b)
    g2d = _sc_scalar_gather(p_lo, idx128)
    final, loss = _finish(g2d, hi, targets)
    return final, loss.reshape(())
